# baseline (speedup 1.0000x reference)
import functools

import jax
import jax.numpy as jnp
from jax import lax
from jax.experimental import pallas as pl
from jax.experimental.pallas import tpu as pltpu
from jax.experimental.pallas import tpu_sc as plsc

_IDX_BLK = 128


@functools.partial(jax.jit, static_argnames=("num_rows", "d", "k", "nch"))
def _sc_gather(idx2d, table, *, num_rows, d, k, nch):
    nw = 32
    ch = k * _IDX_BLK
    bpw = num_rows // nw
    blocks_per_worker = bpw // _IDX_BLK

    mesh = plsc.VectorSubcoreMesh(core_axis_name="c", subcore_axis_name="s")

    @functools.partial(
        pl.kernel,
        mesh=mesh,
        out_type=jax.ShapeDtypeStruct((num_rows, d), jnp.float32),
        scratch_types=[
            pltpu.VMEM((k, _IDX_BLK), jnp.int32),
            pltpu.VMEM((ch, d), jnp.float32),
            pltpu.SemaphoreType.DMA,
        ],
        compiler_params=pltpu.CompilerParams(use_tc_tiling_on_sc=False),
    )
    def body(tokens_hbm, table_hbm, out_hbm, idx_v, rows_v, sem):
        wid = lax.axis_index("s") * 2 + lax.axis_index("c")
        blk0 = wid * blocks_per_worker

        def chunk(g, carry):
            blk = blk0 + g * k
            pltpu.sync_copy(tokens_hbm.at[pl.ds(blk, k)], idx_v)
            copies = [
                pltpu.async_copy(
                    table_hbm.at[idx_v.at[j]],
                    rows_v.at[pl.ds(j * _IDX_BLK, _IDX_BLK)],
                    sem,
                )
                for j in range(k)
            ]
            for c in copies:
                c.wait()
            pltpu.sync_copy(rows_v, out_hbm.at[pl.ds(blk * _IDX_BLK, ch)])
            return carry

        lax.fori_loop(0, nch, chunk, 0)

    return body(idx2d, table)


def kernel(tokens, table):
    b, t = tokens.shape
    num_rows = b * t
    d = table.shape[1]
    idx2d = tokens.astype(jnp.int32).reshape(num_rows // _IDX_BLK, _IDX_BLK)
    k = 8
    nch = num_rows // 32 // (k * _IDX_BLK)
    out = _sc_gather(idx2d, table, num_rows=num_rows, d=d, k=k, nch=nch)
    return out.reshape(b, t, d)
